# baseline (speedup 1.0000x reference)
import functools

import jax
import jax.numpy as jnp
from jax import lax
from jax.experimental import pallas as pl
from jax.experimental.pallas import tpu as pltpu
from jax.experimental.pallas import tpu_sc as plsc


_CHUNK = 16384


def _make_sc_kernel(b, n):
    info = plsc.get_sparse_core_info()
    nc, ns, lanes = info.num_cores, info.num_subcores, info.num_lanes
    nw = nc * ns
    per_w = n // nw
    assert n % nw == 0 and per_w % _CHUNK == 0
    n_chunks = per_w // _CHUNK
    mesh = plsc.VectorSubcoreMesh(core_axis_name="c", subcore_axis_name="s")

    @functools.partial(
        pl.kernel,
        mesh=mesh,
        out_type=jax.ShapeDtypeStruct((b, n), jnp.float32),
        scratch_types=[
            pltpu.VMEM((_CHUNK,), jnp.float32),
            pltpu.VMEM((_CHUNK,), jnp.float32),
        ],
    )
    def k(x_hbm, pos_hbm, out_hbm, pbuf, xbuf):
        wid = lax.axis_index("s") * nc + lax.axis_index("c")
        base = wid * per_w

        def chunk_body(c, carry):
            off = base + c * _CHUNK
            pltpu.sync_copy(pos_hbm.at[pl.ds(off, _CHUNK)], pbuf)
            for bb in range(b):
                pltpu.sync_copy(x_hbm.at[bb, pl.ds(off, _CHUNK)], xbuf)

                def add_body(i, _):
                    sl = pl.ds(i * lanes, lanes)
                    xbuf[sl] = xbuf[sl] + pbuf[sl]
                    return _

                lax.fori_loop(0, _CHUNK // lanes, add_body, 0, unroll=8)
                pltpu.sync_copy(xbuf, out_hbm.at[bb, pl.ds(off, _CHUNK)])
            return carry

        lax.fori_loop(0, n_chunks, chunk_body, 0)

    return k


def kernel(x, pos_table):
    b, s, d = x.shape
    n = s * d
    k = _make_sc_kernel(b, n)
    out = k(x.reshape(b, n), pos_table[:s].reshape(n))
    return out.reshape(b, s, d)


# speedup vs baseline: 1.2124x; 1.2124x over previous
import functools

import jax
import jax.numpy as jnp
from jax import lax
from jax.experimental import pallas as pl
from jax.experimental.pallas import tpu as pltpu
from jax.experimental.pallas import tpu_sc as plsc


_CHUNK = 16384
_NXB = 4
_NPB = 2


def _make_sc_kernel(b, n):
    info = plsc.get_sparse_core_info()
    nc, ns, lanes = info.num_cores, info.num_subcores, info.num_lanes
    nw = nc * ns
    per_w = n // nw
    assert n % nw == 0 and per_w % _CHUNK == 0
    n_chunks = per_w // _CHUNK
    nsteps = n_chunks * b
    mesh = plsc.VectorSubcoreMesh(core_axis_name="c", subcore_axis_name="s")

    @functools.partial(
        pl.kernel,
        mesh=mesh,
        out_type=jax.ShapeDtypeStruct((b, n), jnp.float32),
        scratch_types=(
            [pltpu.VMEM((_CHUNK,), jnp.float32) for _ in range(_NPB + _NXB)]
            + [pltpu.SemaphoreType.DMA for _ in range(3)]
        ),
    )
    def k(x_hbm, pos_hbm, out_hbm, *bufs_and_sems):
        pbufs = bufs_and_sems[:_NPB]
        xbufs = bufs_and_sems[_NPB:_NPB + _NXB]
        psem, lsem, ssem = bufs_and_sems[_NPB + _NXB:]
        wid = lax.axis_index("s") * nc + lax.axis_index("c")
        base = wid * per_w

        def off(c):
            return base + c * _CHUNK

        def load_pos(c):
            return pltpu.async_copy(
                pos_hbm.at[pl.ds(off(c), _CHUNK)], pbufs[c % _NPB], psem)

        def load_x(j):
            c, bb = divmod(j, b)
            return pltpu.async_copy(
                x_hbm.at[bb, pl.ds(off(c), _CHUNK)], xbufs[j % _NXB], lsem)

        pcps = {c: load_pos(c) for c in range(min(_NPB, n_chunks))}
        xcps = {j: load_x(j) for j in range(min(2, nsteps))}
        scps = {}

        for j in range(nsteps):
            c, bb = divmod(j, b)
            buf = xbufs[j % _NXB]
            pbuf = pbufs[c % _NPB]
            if bb == 0:
                pcps.pop(c).wait()
            xcps.pop(j).wait()

            def add_body(i, _, buf=buf, pbuf=pbuf):
                sl = pl.ds(i * lanes, lanes)
                buf[sl] = buf[sl] + pbuf[sl]
                return _

            lax.fori_loop(0, _CHUNK // lanes, add_body, 0, unroll=8)
            if bb == b - 1 and c + _NPB < n_chunks:
                pcps[c + _NPB] = load_pos(c + _NPB)
            scps[j] = pltpu.async_copy(
                buf, out_hbm.at[bb, pl.ds(off(c), _CHUNK)], ssem)
            nxt = j + 2
            if nxt < nsteps:
                drain = nxt - _NXB
                if drain in scps:
                    scps.pop(drain).wait()
                xcps[nxt] = load_x(nxt)

        for j in sorted(scps):
            scps.pop(j).wait()

    return k


def kernel(x, pos_table):
    b, s, d = x.shape
    n = s * d
    k = _make_sc_kernel(b, n)
    out = k(x.reshape(b, n), pos_table[:s].reshape(n))
    return out.reshape(b, s, d)


# speedup vs baseline: 1.7651x; 1.4559x over previous
import functools

import jax
import jax.numpy as jnp
from jax import lax
from jax.experimental import pallas as pl
from jax.experimental.pallas import tpu as pltpu
from jax.experimental.pallas import tpu_sc as plsc


_CHUNK = 8192
_DEPTH = 3


def _make_sc_kernel(b, n):
    info = plsc.get_sparse_core_info()
    nc, ns, lanes = info.num_cores, info.num_subcores, info.num_lanes
    nw = nc * ns
    per_w = n // nw
    assert n % nw == 0 and per_w % _CHUNK == 0
    n_chunks = per_w // _CHUNK
    mesh = plsc.VectorSubcoreMesh(core_axis_name="c", subcore_axis_name="s")

    @functools.partial(
        pl.kernel,
        mesh=mesh,
        out_type=jax.ShapeDtypeStruct((b, n), jnp.float32),
        scratch_types=(
            [pltpu.VMEM((_CHUNK,), jnp.float32)
             for _ in range(_DEPTH * (b + 1))]
            + [pltpu.SemaphoreType.DMA for _ in range(2)]
        ),
    )
    def k(x_hbm, pos_hbm, out_hbm, *bufs_and_sems):
        nbuf = _DEPTH * (b + 1)
        slots = [bufs_and_sems[i * (b + 1):(i + 1) * (b + 1)]
                 for i in range(_DEPTH)]
        lsem, ssem = bufs_and_sems[nbuf:]
        wid = lax.axis_index("s") * nc + lax.axis_index("c")
        base = wid * per_w

        def load_chunk(c):
            slot = slots[c % _DEPTH]
            o = base + c * _CHUNK
            cps = [pltpu.async_copy(pos_hbm.at[pl.ds(o, _CHUNK)], slot[0],
                                    lsem)]
            for bb in range(b):
                cps.append(pltpu.async_copy(
                    x_hbm.at[bb, pl.ds(o, _CHUNK)], slot[1 + bb], lsem))
            return cps

        def store_chunk(c):
            slot = slots[c % _DEPTH]
            o = base + c * _CHUNK
            return [pltpu.async_copy(
                slot[1 + bb], out_hbm.at[bb, pl.ds(o, _CHUNK)], ssem)
                for bb in range(b)]

        loads = {c: load_chunk(c) for c in range(min(2, n_chunks))}
        stores = {}

        for c in range(n_chunks):
            slot = slots[c % _DEPTH]
            pbuf = slot[0]
            for cp in loads.pop(c):
                cp.wait()

            def add_body(i, _, slot=slot, pbuf=pbuf):
                sl = pl.ds(i * lanes, lanes)
                p = pbuf[sl]
                for bb in range(b):
                    plsc.addupdate(slot[1 + bb].at[sl], p)
                return _

            lax.fori_loop(0, _CHUNK // lanes, add_body, 0, unroll=4)
            stores[c] = store_chunk(c)
            if c - 1 in stores:
                for cp in stores.pop(c - 1):
                    cp.wait()
            if c + 2 < n_chunks:
                loads[c + 2] = load_chunk(c + 2)

        for c in sorted(stores):
            for cp in stores.pop(c):
                cp.wait()

    return k


def kernel(x, pos_table):
    b, s, d = x.shape
    n = s * d
    k = _make_sc_kernel(b, n)
    out = k(x.reshape(b, n), pos_table[:s].reshape(n))
    return out.reshape(b, s, d)


# speedup vs baseline: 3.9589x; 2.2429x over previous
import functools

import jax
import jax.numpy as jnp
from jax import lax
from jax.experimental import pallas as pl
from jax.experimental.pallas import tpu as pltpu
from jax.experimental.pallas import tpu_sc as plsc


_ROWS = 8
_DEPTH = 3


def _make_sc_kernel(b, s, d):
    info = plsc.get_sparse_core_info()
    nc, ns, lanes = info.num_cores, info.num_subcores, info.num_lanes
    nw = nc * ns
    rows_w = s // nw
    assert s % nw == 0 and rows_w % _ROWS == 0
    n_slabs = rows_w // _ROWS
    groups = d // lanes
    mesh = plsc.VectorSubcoreMesh(core_axis_name="c", subcore_axis_name="s")

    @functools.partial(
        pl.kernel,
        mesh=mesh,
        out_type=jax.ShapeDtypeStruct((b, s, d), jnp.float32),
        scratch_types=(
            [pltpu.VMEM((_ROWS, d), jnp.float32)
             for _ in range(_DEPTH * (b + 1))]
            + [pltpu.SemaphoreType.DMA for _ in range(2)]
        ),
    )
    def k(x_hbm, pos_hbm, out_hbm, *bufs_and_sems):
        nbuf = _DEPTH * (b + 1)
        slots = [bufs_and_sems[i * (b + 1):(i + 1) * (b + 1)]
                 for i in range(_DEPTH)]
        lsem, ssem = bufs_and_sems[nbuf:]
        wid = lax.axis_index("s") * nc + lax.axis_index("c")
        base = wid * rows_w

        def load_slab(c):
            slot = slots[c % _DEPTH]
            r0 = base + c * _ROWS
            cps = [pltpu.async_copy(pos_hbm.at[pl.ds(r0, _ROWS)], slot[0],
                                    lsem)]
            for bb in range(b):
                cps.append(pltpu.async_copy(
                    x_hbm.at[bb, pl.ds(r0, _ROWS)], slot[1 + bb], lsem))
            return cps

        def store_slab(c):
            slot = slots[c % _DEPTH]
            r0 = base + c * _ROWS
            return [pltpu.async_copy(
                slot[1 + bb], out_hbm.at[bb, pl.ds(r0, _ROWS)], ssem)
                for bb in range(b)]

        loads = {c: load_slab(c) for c in range(min(2, n_slabs))}
        stores = {}

        for c in range(n_slabs):
            slot = slots[c % _DEPTH]
            pbuf = slot[0]
            for cp in loads.pop(c):
                cp.wait()

            def row_body(r, _, slot=slot, pbuf=pbuf):
                def add_body(i, _2, r=r, slot=slot, pbuf=pbuf):
                    sl = pl.ds(i * lanes, lanes)
                    p = pbuf[r, sl]
                    for bb in range(b):
                        plsc.addupdate(slot[1 + bb].at[r, sl], p)
                    return _2

                lax.fori_loop(0, groups, add_body, 0, unroll=4)
                return _

            lax.fori_loop(0, _ROWS, row_body, 0)
            stores[c] = store_slab(c)
            if c - 1 in stores:
                for cp in stores.pop(c - 1):
                    cp.wait()
            if c + 2 < n_slabs:
                loads[c + 2] = load_slab(c + 2)

        for c in sorted(stores):
            for cp in stores.pop(c):
                cp.wait()

    return k


def kernel(x, pos_table):
    b, s, d = x.shape
    k = _make_sc_kernel(b, s, d)
    return k(x, pos_table[:s])
